# baseline (speedup 1.0000x reference)
import functools

import jax
import jax.numpy as jnp
from jax import lax
from jax.experimental import pallas as pl
from jax.experimental.pallas import tpu as pltpu
from jax.experimental.pallas import tpu_sc as plsc

_VOCAB = 1000
_D = 16
_B = 16384
_F = 26

_NC = 2
_NS = 16
_NW = _NC * _NS

_BBLK = 128
_NBLK = _B // _BBLK
_BLK_W = _NBLK // _NW
_LANES = 16


def _recip_norm(acc):
    i = lax.bitcast_convert_type(acc, jnp.int32)
    i = jnp.int32(0x5F3759DF) - (i >> 1)
    y = lax.bitcast_convert_type(i, jnp.float32)
    for _ in range(2):
        y = y * (jnp.float32(1.5) - jnp.float32(0.5) * acc * y * y)
    norm = acc * y
    return jnp.float32(1.0) / jnp.maximum(norm, jnp.float32(1e-12))


def _body(table_hbm, idx_hbm, out_hbm, table_v, idx_v, out_st, gsem):
    wid = lax.axis_index("s") * _NC + lax.axis_index("c")

    pltpu.sync_copy(table_hbm, table_v)


    def do_block(cb, _):
        blk = wid * _BLK_W + cb
        pltpu.sync_copy(idx_hbm.at[:, pl.ds(blk * _BBLK, _BBLK)], idx_v)

        def do_group(bb, _):
            iv = [idx_v[f, pl.ds(bb * _LANES, _LANES)] for f in range(_F)]
            fv = [v * _D for v in iv]
            xf = [v & 15 for v in iv]

            def do_dim(d, _):
                gs = [plsc.load_gather(table_v, [fv[f] + (xf[f] ^ d)])
                      for f in range(_F)]
                sq = [g * g for g in gs]
                while len(sq) > 1:
                    sq = [sq[i] + sq[i + 1] for i in range(0, len(sq) - 1, 2)]                          + ([sq[-1]] if len(sq) % 2 else [])
                recip = _recip_norm(sq[0])
                dhi = d >> 3
                dlo = d & 7
                for f in range(_F):
                    out_st[f, dhi, dlo, pl.ds(bb * _LANES, _LANES)] = (
                        gs[f] * recip)
                return _

            lax.fori_loop(0, _D, do_dim, None)
            return _

        lax.fori_loop(0, _BBLK // _LANES, do_group, None)
        pltpu.sync_copy(out_st, out_hbm.at[:, :, blk])
        return _

    lax.fori_loop(0, _BLK_W, do_block, None)


def kernel(user_features, embedding_table):
    mesh = plsc.VectorSubcoreMesh(
        core_axis_name="c", subcore_axis_name="s",
        num_cores=_NC, num_subcores=_NS)
    run = functools.partial(
        pl.kernel,
        out_type=jax.ShapeDtypeStruct((_F, _D // 8, _NBLK, 8, _BBLK),
                                      jnp.float32),
        mesh=mesh,
        scratch_types=[
            pltpu.VMEM((_VOCAB * _D,), jnp.float32),
            pltpu.VMEM((_F, _BBLK), jnp.int32),
            pltpu.VMEM((_F, _D // 8, 8, _BBLK), jnp.float32),
            pltpu.SemaphoreType.DMA,
        ],
        compiler_params=pltpu.CompilerParams(use_tc_tiling_on_sc=False, needs_layout_passes=False),
    )(_body)
    idx_t = user_features.T
    rr = jnp.arange(_VOCAB, dtype=jnp.int32)[:, None] & 15
    jj = jnp.arange(_D, dtype=jnp.int32)[None, :]
    table_sw = jnp.take_along_axis(embedding_table, jj ^ rr, axis=1)
    out5 = run(table_sw.reshape(_VOCAB * _D), idx_t)
    out = jnp.transpose(out5, (2, 4, 0, 1, 3))
    return out.reshape(_B, _F, _D)


# speedup vs baseline: 1.0191x; 1.0191x over previous
import functools

import jax
import jax.numpy as jnp
from jax import lax
from jax.experimental import pallas as pl
from jax.experimental.pallas import tpu as pltpu
from jax.experimental.pallas import tpu_sc as plsc

_VOCAB = 1000
_D = 16
_B = 16384
_F = 26

_NC = 2
_NS = 16
_NW = _NC * _NS

_BBLK = 128
_NBLK = _B // _BBLK
_BLK_W = _NBLK // _NW
_LANES = 16


def _recip_norm(acc):
    i = lax.bitcast_convert_type(acc, jnp.int32)
    i = jnp.int32(0x5F3759DF) - (i >> 1)
    y = lax.bitcast_convert_type(i, jnp.float32)
    for _ in range(2):
        y = y * (jnp.float32(1.5) - jnp.float32(0.5) * acc * y * y)
    norm = acc * y
    return jnp.float32(1.0) / jnp.maximum(norm, jnp.float32(1e-12))


def _body(table_hbm, idx_hbm, out_hbm, table_v, idx_v, out_st, gsem):
    wid = lax.axis_index("s") * _NC + lax.axis_index("c")

    pltpu.sync_copy(table_hbm, table_v)

    iota16 = lax.iota(jnp.int32, _LANES)

    def swizzle_row(r, _):
        base = r * _D
        sh = plsc.load_gather(table_v, [base + (iota16 ^ (r & 15))])
        table_v[pl.ds(base, _D)] = sh
        return _

    lax.fori_loop(0, _VOCAB, swizzle_row, None)

    def do_block(cb, _):
        blk = wid * _BLK_W + cb
        pltpu.sync_copy(idx_hbm.at[:, pl.ds(blk * _BBLK, _BBLK)], idx_v)

        def do_group(bb, _):
            iv = [idx_v[f, pl.ds(bb * _LANES, _LANES)] for f in range(_F)]
            fv = [v * _D for v in iv]
            xf = [v & 15 for v in iv]

            def do_dim(d, _):
                gs = [plsc.load_gather(table_v, [fv[f] + (xf[f] ^ d)])
                      for f in range(_F)]
                sq = [g * g for g in gs]
                while len(sq) > 1:
                    sq = [sq[i] + sq[i + 1] for i in range(0, len(sq) - 1, 2)]                          + ([sq[-1]] if len(sq) % 2 else [])
                recip = _recip_norm(sq[0])
                dhi = d >> 3
                dlo = d & 7
                for f in range(_F):
                    out_st[f, dhi, dlo, pl.ds(bb * _LANES, _LANES)] = (
                        gs[f] * recip)
                return _

            lax.fori_loop(0, _D, do_dim, None)
            return _

        lax.fori_loop(0, _BBLK // _LANES, do_group, None)
        pltpu.sync_copy(out_st, out_hbm.at[:, :, blk])
        return _

    lax.fori_loop(0, _BLK_W, do_block, None)


def kernel(user_features, embedding_table):
    mesh = plsc.VectorSubcoreMesh(
        core_axis_name="c", subcore_axis_name="s",
        num_cores=_NC, num_subcores=_NS)
    run = functools.partial(
        pl.kernel,
        out_type=jax.ShapeDtypeStruct((_F, _D // 8, _NBLK, 8, _BBLK),
                                      jnp.float32),
        mesh=mesh,
        scratch_types=[
            pltpu.VMEM((_VOCAB * _D,), jnp.float32),
            pltpu.VMEM((_F, _BBLK), jnp.int32),
            pltpu.VMEM((_F, _D // 8, 8, _BBLK), jnp.float32),
            pltpu.SemaphoreType.DMA,
        ],
        compiler_params=pltpu.CompilerParams(use_tc_tiling_on_sc=False, needs_layout_passes=False),
    )(_body)
    idx_t = user_features.T
    out5 = run(embedding_table.reshape(_VOCAB * _D), idx_t)
    out = jnp.transpose(out5, (2, 4, 0, 1, 3))
    return out.reshape(_B, _F, _D)
